# baseline (speedup 1.0000x reference)
import functools

import jax
import jax.numpy as jnp
from jax import lax
from jax.experimental import pallas as pl
from jax.experimental.pallas import tpu as pltpu
from jax.experimental.pallas import tpu_sc as plsc

_VOCAB = 1000000
_D = 64
_B = 1024
_L = 200
_MASK_ID = 103

_N = _B * _L
_CH = 128
_NW = 32
_PER_W = _N // _NW
_NCH = _PER_W // _CH


def _body(ids_hbm, p_hbm, table_hbm, out_hbm, idx_v, p_v, rows_v, out_v,
          midx_v, mrow_v, sem):
    nc = 2
    wid = lax.axis_index("s") * nc + lax.axis_index("c")
    base = wid * _PER_W

    pltpu.sync_copy(ids_hbm.at[pl.ds(base, _PER_W)], idx_v)
    pltpu.sync_copy(p_hbm.at[pl.ds(base, _PER_W)], p_v)

    midx_v[...] = jnp.full((16,), _MASK_ID, jnp.int32)
    pltpu.async_copy(table_hbm.at[midx_v], mrow_v, sem).wait()
    m = [mrow_v[0, pl.ds(k * 16, 16)] for k in range(4)]

    def chunk(g, _):
        pltpu.async_copy(
            table_hbm.at[idx_v.at[pl.ds(g * _CH, _CH)]], rows_v, sem
        ).wait()

        def group(j, _):
            pv = p_v[pl.ds(g * _CH + j * 16, 16)]
            qv = 1.0 - pv
            for k in range(16):
                p = pv[k]
                q = qv[k]
                i = j * 16 + k
                for kk in range(4):
                    out_v[pl.ds(i * _D + kk * 16, 16)] = (
                        p * m[kk] + q * rows_v[i, pl.ds(kk * 16, 16)]
                    )
            return 0

        lax.fori_loop(0, _CH // 16, group, 0)
        pltpu.sync_copy(
            out_v, out_hbm.at[pl.ds((base + g * _CH) * _D, _CH * _D)]
        )
        return 0

    lax.fori_loop(0, _NCH, chunk, 0)


@jax.jit
def _soft_mask_sc(ids_flat, p_flat, table):
    mesh = plsc.VectorSubcoreMesh(core_axis_name="c", subcore_axis_name="s")
    f = functools.partial(
        pl.kernel,
        out_type=jax.ShapeDtypeStruct((_N * _D,), jnp.float32),
        mesh=mesh,
        compiler_params=pltpu.CompilerParams(use_tc_tiling_on_sc=False),
        scratch_types=[
            pltpu.VMEM((_PER_W,), jnp.int32),
            pltpu.VMEM((_PER_W,), jnp.float32),
            pltpu.VMEM((_CH, _D), jnp.float32),
            pltpu.VMEM((_CH * _D,), jnp.float32),
            pltpu.VMEM((16,), jnp.int32),
            pltpu.VMEM((16, _D), jnp.float32),
            pltpu.SemaphoreType.DMA,
        ],
    )(_body)
    return f(ids_flat, p_flat, table)


def kernel(input_ids, detect_prob, table):
    ids_flat = input_ids.reshape(_N)
    p_flat = detect_prob.reshape(_N)
    out = _soft_mask_sc(ids_flat, p_flat, table)
    return out.reshape(_B, _L, _D)
